# Initial kernel scaffold; baseline (speedup 1.0000x reference)
import jax
import jax.numpy as jnp
from jax.experimental import pallas as pl


def kernel(price, item_id, cate_id, hist_item_id, hist_cate_id, hist_item_id_length, hist_cate_id_length, emb_item, emb_cate, emb_hist_item, emb_hist_cate, att_W1, att_b1, att_a1, att_W2, att_b2, att_a2, att_Wo, att_bo, mlp_W1, mlp_b1, mlp_g1, mlp_be1, mlp_a1, mlp_W2, mlp_b2, mlp_g2, mlp_be2, mlp_a2, fin_W, fin_b):
    raise NotImplementedError("write your pallas kernel here")


# speedup vs baseline: 1.8815x; 1.8815x over previous
import functools

import jax
import jax.numpy as jnp
from jax import lax
from jax.experimental import pallas as pl
from jax.experimental.pallas import tpu as pltpu
from jax.experimental.pallas import tpu_sc as plsc

B = 4096
T = 200
D = 32
NW = 32
CH = 1600
BB = 64

PREC = jax.lax.Precision.HIGHEST


def _sc_gather_all(emb_item, emb_cate, emb_hist_item, emb_hist_cate,
                   item_id, cate_id, hist_item_flat, hist_cate_flat):
    n = hist_item_flat.shape[0]
    per_w = n // NW
    n_ch = per_w // CH
    qn = B // NW

    mesh = plsc.VectorSubcoreMesh(core_axis_name="c", subcore_axis_name="s")

    @functools.partial(
        pl.kernel,
        out_type=[jax.ShapeDtypeStruct((n, D), jnp.float32),
                  jax.ShapeDtypeStruct((n, D), jnp.float32),
                  jax.ShapeDtypeStruct((B, D), jnp.float32),
                  jax.ShapeDtypeStruct((B, D), jnp.float32)],
        mesh=mesh,
        scratch_types=[pltpu.VMEM((CH,), jnp.int32),
                       pltpu.VMEM((CH, D), jnp.float32),
                       pltpu.VMEM((qn,), jnp.int32),
                       pltpu.VMEM((qn, D), jnp.float32),
                       pltpu.SemaphoreType.DMA],
        compiler_params=pltpu.CompilerParams(use_tc_tiling_on_sc=False),
    )
    def gather_kernel(ei_hbm, ec_hbm, ehi_hbm, ehc_hbm,
                      ii_hbm, ci_hbm, hi_hbm, hc_hbm,
                      ki_out, kc_out, qi_out, qc_out,
                      idx_v, rows_v, qidx_v, qrows_v, sem):
        wid = lax.axis_index("s") * 2 + lax.axis_index("c")
        base = wid * per_w

        @pl.loop(0, n_ch)
        def _(c):
            off = base + c * CH
            pltpu.sync_copy(hi_hbm.at[pl.ds(off, CH)], idx_v)
            pltpu.async_copy(ehi_hbm.at[idx_v], rows_v, sem).wait()
            pltpu.sync_copy(rows_v, ki_out.at[pl.ds(off, CH)])
            pltpu.sync_copy(hc_hbm.at[pl.ds(off, CH)], idx_v)
            pltpu.async_copy(ehc_hbm.at[idx_v], rows_v, sem).wait()
            pltpu.sync_copy(rows_v, kc_out.at[pl.ds(off, CH)])

        qbase = wid * qn
        pltpu.sync_copy(ii_hbm.at[pl.ds(qbase, qn)], qidx_v)
        pltpu.async_copy(ei_hbm.at[qidx_v], qrows_v, sem).wait()
        pltpu.sync_copy(qrows_v, qi_out.at[pl.ds(qbase, qn)])
        pltpu.sync_copy(ci_hbm.at[pl.ds(qbase, qn)], qidx_v)
        pltpu.async_copy(ec_hbm.at[qidx_v], qrows_v, sem).wait()
        pltpu.sync_copy(qrows_v, qc_out.at[pl.ds(qbase, qn)])

    return gather_kernel(emb_item, emb_cate, emb_hist_item, emb_hist_cate,
                         item_id, cate_id, hist_item_flat, hist_cate_flat)


def _att_body(qi_ref, qc_ref, ki_ref, kc_ref, len_ref, wq_ref, wkp_ref,
              b1_ref, w2_ref, b2_ref, wo_ref, sc_ref, out_ref):
    a1 = sc_ref[0, 0]
    a2 = sc_ref[0, 1]
    bo = sc_ref[0, 2]
    q = jnp.concatenate([qi_ref[...], qc_ref[...]], axis=1)
    keys = jnp.concatenate([ki_ref[...], kc_ref[...]], axis=1)
    k3 = keys.reshape(BB, T, 2 * D)
    qk = (k3 * q[:, None, :]).reshape(BB * T, 2 * D)
    m = jnp.concatenate([keys, qk], axis=1)
    tq = jnp.dot(q, wq_ref[...], preferred_element_type=jnp.float32,
                 precision=PREC) + b1_ref[...]
    h1 = jnp.dot(m, wkp_ref[...], preferred_element_type=jnp.float32,
                 precision=PREC)
    h1 = h1.reshape(BB, T, 80) + tq[:, None, :]
    h1 = jnp.where(h1 > 0, h1, a1 * h1).reshape(BB * T, 80)
    h2 = jnp.dot(h1, w2_ref[...], preferred_element_type=jnp.float32,
                 precision=PREC) + b2_ref[...]
    h2 = jnp.where(h2 > 0, h2, a2 * h2)
    s3 = h2.reshape(BB, T, 40) * wo_ref[...][None, :, :]
    scores = (jnp.sum(s3, axis=2) + bo) * jnp.float32(0.125)
    lens = len_ref[0, 0, :]
    tidx = lax.broadcasted_iota(jnp.int32, (BB, T), 1)
    scores = jnp.where(tidx < lens[:, None], scores, jnp.float32(-1e9))
    mx = jnp.max(scores, axis=1, keepdims=True)
    e = jnp.exp(scores - mx)
    attn = e / jnp.sum(e, axis=1, keepdims=True)
    out_ref[...] = jnp.sum(k3 * attn[:, :, None], axis=1)


def _attention(q_item, q_cate, k_item, k_cate, len3,
               wq, wkp, b1, w2, b2, wo_t, sc):
    rep = lambda shape: pl.BlockSpec(shape, lambda i: tuple(0 for _ in shape))
    return pl.pallas_call(
        _att_body,
        grid=(B // BB,),
        in_specs=[
            pl.BlockSpec((BB, D), lambda i: (i, 0)),
            pl.BlockSpec((BB, D), lambda i: (i, 0)),
            pl.BlockSpec((BB * T, D), lambda i: (i, 0)),
            pl.BlockSpec((BB * T, D), lambda i: (i, 0)),
            pl.BlockSpec((1, 1, BB), lambda i: (i, 0, 0)),
            rep((2 * D, 80)),
            rep((4 * D, 80)),
            rep((1, 80)),
            rep((80, 40)),
            rep((1, 40)),
            rep((1, 40)),
            rep((1, 8)),
        ],
        out_specs=pl.BlockSpec((BB, 2 * D), lambda i: (i, 0)),
        out_shape=jax.ShapeDtypeStruct((B, 2 * D), jnp.float32),
    )(q_item, q_cate, k_item, k_cate, len3, wq, wkp, b1, w2, b2, wo_t, sc)


def _mlp_body(pr_ref, qi_ref, qc_ref, ao_ref, w1_ref, b1_ref, g1_ref, be1_ref,
              w2_ref, b2_ref, g2_ref, be2_ref, fw_ref, sc_ref, out_ref):
    a1 = sc_ref[0, 0]
    a2 = sc_ref[0, 1]
    fb = sc_ref[0, 2]
    x = jnp.concatenate([pr_ref[...], qi_ref[...], qc_ref[...], ao_ref[...]],
                        axis=1)
    h = jnp.dot(x, w1_ref[...], preferred_element_type=jnp.float32,
                precision=PREC) + b1_ref[...]
    mu = jnp.mean(h, axis=0, keepdims=True)
    var = jnp.mean((h - mu) ** 2, axis=0, keepdims=True)
    h = g1_ref[...] * (h - mu) / jnp.sqrt(var + 1e-5) + be1_ref[...]
    h = jnp.where(h > 0, h, a1 * h)
    h = jnp.dot(h, w2_ref[...], preferred_element_type=jnp.float32,
                precision=PREC) + b2_ref[...]
    mu = jnp.mean(h, axis=0, keepdims=True)
    var = jnp.mean((h - mu) ** 2, axis=0, keepdims=True)
    h = g2_ref[...] * (h - mu) / jnp.sqrt(var + 1e-5) + be2_ref[...]
    h = jnp.where(h > 0, h, a2 * h)
    o = jnp.dot(h, fw_ref[...], preferred_element_type=jnp.float32,
                precision=PREC) + fb
    out_ref[...] = jax.nn.sigmoid(o)


def _mlp(price2, q_item, q_cate, att_out,
         w1, b1, g1, be1, w2, b2, g2, be2, fw, sc):
    return pl.pallas_call(
        _mlp_body,
        out_shape=jax.ShapeDtypeStruct((B, 1), jnp.float32),
    )(price2, q_item, q_cate, att_out, w1, b1, g1, be1, w2, b2, g2, be2,
      fw, sc)


def kernel(price, item_id, cate_id, hist_item_id, hist_cate_id,
           hist_item_id_length, hist_cate_id_length,
           emb_item, emb_cate, emb_hist_item, emb_hist_cate,
           att_W1, att_b1, att_a1, att_W2, att_b2, att_a2, att_Wo, att_bo,
           mlp_W1, mlp_b1, mlp_g1, mlp_be1, mlp_a1,
           mlp_W2, mlp_b2, mlp_g2, mlp_be2, mlp_a2,
           fin_W, fin_b):
    k_item, k_cate, q_item, q_cate = _sc_gather_all(
        emb_item, emb_cate, emb_hist_item, emb_hist_cate,
        item_id, cate_id,
        hist_item_id.reshape(-1), hist_cate_id.reshape(-1))

    wq = att_W1[0:2 * D] + att_W1[4 * D:6 * D]
    wk = att_W1[2 * D:4 * D] - att_W1[4 * D:6 * D]
    wp = att_W1[6 * D:8 * D]
    wkp = jnp.concatenate([wk, wp], axis=0)

    keys_len = jnp.minimum(hist_item_id_length, hist_cate_id_length)
    len3 = keys_len.astype(jnp.int32).reshape(B // BB, 1, BB)
    sc_att = jnp.stack([att_a1, att_a2, att_bo[0]] + [jnp.float32(0)] * 5)
    att_out = _attention(q_item, q_cate, k_item, k_cate, len3,
                         wq, wkp, att_b1.reshape(1, 80),
                         att_W2, att_b2.reshape(1, 40),
                         att_Wo.reshape(1, 40), sc_att.reshape(1, 8))

    sc_mlp = jnp.stack([mlp_a1, mlp_a2, fin_b[0]] + [jnp.float32(0)] * 5)
    return _mlp(price.reshape(B, 1), q_item, q_cate, att_out,
                mlp_W1, mlp_b1.reshape(1, 200), mlp_g1.reshape(1, 200),
                mlp_be1.reshape(1, 200), mlp_W2, mlp_b2.reshape(1, 80),
                mlp_g2.reshape(1, 80), mlp_be2.reshape(1, 80),
                fin_W, sc_mlp.reshape(1, 8))


# speedup vs baseline: 2.7662x; 1.4702x over previous
import functools

import jax
import jax.numpy as jnp
from jax import lax
from jax.experimental import pallas as pl
from jax.experimental.pallas import tpu as pltpu
from jax.experimental.pallas import tpu_sc as plsc

B = 4096
T = 200
D = 32
NW = 32
CH = 1600
BB = 64

PREC = jax.lax.Precision.DEFAULT


def _sc_gather_all(emb_item, emb_cate, emb_hist_item, emb_hist_cate,
                   item_id, cate_id, hist_item_flat, hist_cate_flat):
    n = hist_item_flat.shape[0]
    per_w = n // NW
    n_ch = per_w // CH
    qn = B // NW

    mesh = plsc.VectorSubcoreMesh(core_axis_name="c", subcore_axis_name="s")

    @functools.partial(
        pl.kernel,
        out_type=[jax.ShapeDtypeStruct((n, D), jnp.float32),
                  jax.ShapeDtypeStruct((n, D), jnp.float32),
                  jax.ShapeDtypeStruct((B, D), jnp.float32),
                  jax.ShapeDtypeStruct((B, D), jnp.float32)],
        mesh=mesh,
        scratch_types=[pltpu.VMEM((CH,), jnp.int32),
                       pltpu.VMEM((CH, D), jnp.float32),
                       pltpu.VMEM((qn,), jnp.int32),
                       pltpu.VMEM((qn, D), jnp.float32),
                       pltpu.SemaphoreType.DMA],
        compiler_params=pltpu.CompilerParams(use_tc_tiling_on_sc=False),
    )
    def gather_kernel(ei_hbm, ec_hbm, ehi_hbm, ehc_hbm,
                      ii_hbm, ci_hbm, hi_hbm, hc_hbm,
                      ki_out, kc_out, qi_out, qc_out,
                      idx_v, rows_v, qidx_v, qrows_v, sem):
        wid = lax.axis_index("s") * 2 + lax.axis_index("c")
        base = wid * per_w

        @pl.loop(0, n_ch)
        def _(c):
            off = base + c * CH
            pltpu.sync_copy(hi_hbm.at[pl.ds(off, CH)], idx_v)
            pltpu.async_copy(ehi_hbm.at[idx_v], rows_v, sem).wait()
            pltpu.sync_copy(rows_v, ki_out.at[pl.ds(off, CH)])
            pltpu.sync_copy(hc_hbm.at[pl.ds(off, CH)], idx_v)
            pltpu.async_copy(ehc_hbm.at[idx_v], rows_v, sem).wait()
            pltpu.sync_copy(rows_v, kc_out.at[pl.ds(off, CH)])

        qbase = wid * qn
        pltpu.sync_copy(ii_hbm.at[pl.ds(qbase, qn)], qidx_v)
        pltpu.async_copy(ei_hbm.at[qidx_v], qrows_v, sem).wait()
        pltpu.sync_copy(qrows_v, qi_out.at[pl.ds(qbase, qn)])
        pltpu.sync_copy(ci_hbm.at[pl.ds(qbase, qn)], qidx_v)
        pltpu.async_copy(ec_hbm.at[qidx_v], qrows_v, sem).wait()
        pltpu.sync_copy(qrows_v, qc_out.at[pl.ds(qbase, qn)])

    return gather_kernel(emb_item, emb_cate, emb_hist_item, emb_hist_cate,
                         item_id, cate_id, hist_item_flat, hist_cate_flat)


def _att_body(qi_ref, qc_ref, ki_ref, kc_ref, len_ref, wq_ref, wkp_ref,
              b1_ref, w2_ref, b2_ref, wo_ref, sc_ref, out_ref):
    a1 = sc_ref[0, 0]
    a2 = sc_ref[0, 1]
    bo = sc_ref[0, 2]
    q = jnp.concatenate([qi_ref[...], qc_ref[...]], axis=1)
    keys = jnp.concatenate([ki_ref[...], kc_ref[...]], axis=1)
    k3 = keys.reshape(BB, T, 2 * D)
    qk = (k3 * q[:, None, :]).reshape(BB * T, 2 * D)
    m = jnp.concatenate([keys, qk], axis=1)
    tq = jnp.dot(q, wq_ref[...], preferred_element_type=jnp.float32,
                 precision=PREC) + b1_ref[...]
    h1 = jnp.dot(m, wkp_ref[...], preferred_element_type=jnp.float32,
                 precision=PREC)
    h1 = h1.reshape(BB, T, 80) + tq[:, None, :]
    h1 = jnp.where(h1 > 0, h1, a1 * h1).reshape(BB * T, 80)
    h2 = jnp.dot(h1, w2_ref[...], preferred_element_type=jnp.float32,
                 precision=PREC) + b2_ref[...]
    h2 = jnp.where(h2 > 0, h2, a2 * h2)
    s3 = h2.reshape(BB, T, 40) * wo_ref[...][None, :, :]
    scores = (jnp.sum(s3, axis=2) + bo) * jnp.float32(0.125)
    lens = len_ref[0, 0, :]
    tidx = lax.broadcasted_iota(jnp.int32, (BB, T), 1)
    scores = jnp.where(tidx < lens[:, None], scores, jnp.float32(-1e9))
    mx = jnp.max(scores, axis=1, keepdims=True)
    e = jnp.exp(scores - mx)
    attn = e / jnp.sum(e, axis=1, keepdims=True)
    out_ref[...] = jnp.sum(k3 * attn[:, :, None], axis=1)


def _attention(q_item, q_cate, k_item, k_cate, len3,
               wq, wkp, b1, w2, b2, wo_t, sc):
    rep = lambda shape: pl.BlockSpec(shape, lambda i: tuple(0 for _ in shape))
    return pl.pallas_call(
        _att_body,
        grid=(B // BB,),
        in_specs=[
            pl.BlockSpec((BB, D), lambda i: (i, 0)),
            pl.BlockSpec((BB, D), lambda i: (i, 0)),
            pl.BlockSpec((BB * T, D), lambda i: (i, 0)),
            pl.BlockSpec((BB * T, D), lambda i: (i, 0)),
            pl.BlockSpec((1, 1, BB), lambda i: (i, 0, 0)),
            rep((2 * D, 80)),
            rep((4 * D, 80)),
            rep((1, 80)),
            rep((80, 40)),
            rep((1, 40)),
            rep((1, 40)),
            rep((1, 8)),
        ],
        out_specs=pl.BlockSpec((BB, 2 * D), lambda i: (i, 0)),
        out_shape=jax.ShapeDtypeStruct((B, 2 * D), jnp.float32),
    )(q_item, q_cate, k_item, k_cate, len3, wq, wkp, b1, w2, b2, wo_t, sc)


def _mlp_body(pr_ref, qi_ref, qc_ref, ao_ref, w1_ref, b1_ref, g1_ref, be1_ref,
              w2_ref, b2_ref, g2_ref, be2_ref, fw_ref, sc_ref, out_ref):
    a1 = sc_ref[0, 0]
    a2 = sc_ref[0, 1]
    fb = sc_ref[0, 2]
    x = jnp.concatenate([pr_ref[...], qi_ref[...], qc_ref[...], ao_ref[...]],
                        axis=1)
    h = jnp.dot(x, w1_ref[...], preferred_element_type=jnp.float32,
                precision=PREC) + b1_ref[...]
    mu = jnp.mean(h, axis=0, keepdims=True)
    var = jnp.mean((h - mu) ** 2, axis=0, keepdims=True)
    h = g1_ref[...] * (h - mu) / jnp.sqrt(var + 1e-5) + be1_ref[...]
    h = jnp.where(h > 0, h, a1 * h)
    h = jnp.dot(h, w2_ref[...], preferred_element_type=jnp.float32,
                precision=PREC) + b2_ref[...]
    mu = jnp.mean(h, axis=0, keepdims=True)
    var = jnp.mean((h - mu) ** 2, axis=0, keepdims=True)
    h = g2_ref[...] * (h - mu) / jnp.sqrt(var + 1e-5) + be2_ref[...]
    h = jnp.where(h > 0, h, a2 * h)
    o = jnp.dot(h, fw_ref[...], preferred_element_type=jnp.float32,
                precision=PREC) + fb
    out_ref[...] = jax.nn.sigmoid(o)


def _mlp(price2, q_item, q_cate, att_out,
         w1, b1, g1, be1, w2, b2, g2, be2, fw, sc):
    return pl.pallas_call(
        _mlp_body,
        out_shape=jax.ShapeDtypeStruct((B, 1), jnp.float32),
    )(price2, q_item, q_cate, att_out, w1, b1, g1, be1, w2, b2, g2, be2,
      fw, sc)


def kernel(price, item_id, cate_id, hist_item_id, hist_cate_id,
           hist_item_id_length, hist_cate_id_length,
           emb_item, emb_cate, emb_hist_item, emb_hist_cate,
           att_W1, att_b1, att_a1, att_W2, att_b2, att_a2, att_Wo, att_bo,
           mlp_W1, mlp_b1, mlp_g1, mlp_be1, mlp_a1,
           mlp_W2, mlp_b2, mlp_g2, mlp_be2, mlp_a2,
           fin_W, fin_b):
    k_item, k_cate, q_item, q_cate = _sc_gather_all(
        emb_item, emb_cate, emb_hist_item, emb_hist_cate,
        item_id, cate_id,
        hist_item_id.reshape(-1), hist_cate_id.reshape(-1))

    wq = att_W1[0:2 * D] + att_W1[4 * D:6 * D]
    wk = att_W1[2 * D:4 * D] - att_W1[4 * D:6 * D]
    wp = att_W1[6 * D:8 * D]
    wkp = jnp.concatenate([wk, wp], axis=0)

    keys_len = jnp.minimum(hist_item_id_length, hist_cate_id_length)
    len3 = keys_len.astype(jnp.int32).reshape(B // BB, 1, BB)
    sc_att = jnp.stack([att_a1, att_a2, att_bo[0]] + [jnp.float32(0)] * 5)
    att_out = _attention(q_item, q_cate, k_item, k_cate, len3,
                         wq, wkp, att_b1.reshape(1, 80),
                         att_W2, att_b2.reshape(1, 40),
                         att_Wo.reshape(1, 40), sc_att.reshape(1, 8))

    sc_mlp = jnp.stack([mlp_a1, mlp_a2, fin_b[0]] + [jnp.float32(0)] * 5)
    return _mlp(price.reshape(B, 1), q_item, q_cate, att_out,
                mlp_W1, mlp_b1.reshape(1, 200), mlp_g1.reshape(1, 200),
                mlp_be1.reshape(1, 200), mlp_W2, mlp_b2.reshape(1, 80),
                mlp_g2.reshape(1, 80), mlp_be2.reshape(1, 80),
                fin_W, sc_mlp.reshape(1, 8))


# speedup vs baseline: 3.1798x; 1.1495x over previous
import functools

import jax
import jax.numpy as jnp
from jax import lax
from jax.experimental import pallas as pl
from jax.experimental.pallas import tpu as pltpu
from jax.experimental.pallas import tpu_sc as plsc

B = 4096
T = 200
H = T // 2
D = 32
NW = 32
CH = 1600
CHH = CH // 2
BB = 64

PREC = jax.lax.Precision.DEFAULT


def _sc_gather_all(emb_item, emb_cate, emb_hist_item, emb_hist_cate,
                   item_id, cate_id, hist_item_perm, hist_cate_perm):
    n = hist_item_perm.shape[0]
    per_w = n // NW
    n_ch = per_w // CH
    qn = B // NW

    mesh = plsc.VectorSubcoreMesh(core_axis_name="c", subcore_axis_name="s")

    @functools.partial(
        pl.kernel,
        out_type=[jax.ShapeDtypeStruct((n // 2, 4 * D), jnp.float32),
                  jax.ShapeDtypeStruct((B, D), jnp.float32),
                  jax.ShapeDtypeStruct((B, D), jnp.float32)],
        mesh=mesh,
        scratch_types=[pltpu.VMEM((CH,), jnp.int32),
                       pltpu.VMEM((CH, D), jnp.float32),
                       pltpu.VMEM((qn,), jnp.int32),
                       pltpu.VMEM((qn, D), jnp.float32),
                       pltpu.SemaphoreType.DMA],
        compiler_params=pltpu.CompilerParams(use_tc_tiling_on_sc=False),
    )
    def gather_kernel(ei_hbm, ec_hbm, ehi_hbm, ehc_hbm,
                      ii_hbm, ci_hbm, hi_hbm, hc_hbm,
                      kp_out, qi_out, qc_out,
                      idx_v, rows_v, qidx_v, qrows_v, sem):
        wid = lax.axis_index("s") * 2 + lax.axis_index("c")
        base = wid * per_w

        @pl.loop(0, n_ch)
        def _(c):
            off = base + c * CH
            row0 = off // 2
            pltpu.sync_copy(hi_hbm.at[pl.ds(off, CH)], idx_v)
            pltpu.async_copy(ehi_hbm.at[idx_v], rows_v, sem).wait()
            pltpu.sync_copy(rows_v.at[pl.ds(0, CHH)],
                            kp_out.at[pl.ds(row0, CHH), pl.ds(0, D)])
            pltpu.sync_copy(rows_v.at[pl.ds(CHH, CHH)],
                            kp_out.at[pl.ds(row0, CHH), pl.ds(2 * D, D)])
            pltpu.sync_copy(hc_hbm.at[pl.ds(off, CH)], idx_v)
            pltpu.async_copy(ehc_hbm.at[idx_v], rows_v, sem).wait()
            pltpu.sync_copy(rows_v.at[pl.ds(0, CHH)],
                            kp_out.at[pl.ds(row0, CHH), pl.ds(D, D)])
            pltpu.sync_copy(rows_v.at[pl.ds(CHH, CHH)],
                            kp_out.at[pl.ds(row0, CHH), pl.ds(3 * D, D)])

        qbase = wid * qn
        pltpu.sync_copy(ii_hbm.at[pl.ds(qbase, qn)], qidx_v)
        pltpu.async_copy(ei_hbm.at[qidx_v], qrows_v, sem).wait()
        pltpu.sync_copy(qrows_v, qi_out.at[pl.ds(qbase, qn)])
        pltpu.sync_copy(ci_hbm.at[pl.ds(qbase, qn)], qidx_v)
        pltpu.async_copy(ec_hbm.at[qidx_v], qrows_v, sem).wait()
        pltpu.sync_copy(qrows_v, qc_out.at[pl.ds(qbase, qn)])

    return gather_kernel(emb_item, emb_cate, emb_hist_item, emb_hist_cate,
                         item_id, cate_id, hist_item_perm, hist_cate_perm)


def _parity_perm(ids):
    return ids.reshape(-1).reshape(B * T // CH, CHH, 2).transpose(0, 2, 1).reshape(-1)


def _att_body(qi_ref, qc_ref, kp_ref, len_ref, wq_ref, b1_ref, wbk_ref,
              wbp_ref, w2b_ref, b2_ref, woe_ref, woo_ref, sc_ref, out_ref):
    a1 = sc_ref[0, 0]
    a2 = sc_ref[0, 1]
    bo = sc_ref[0, 2]
    q = jnp.concatenate([qi_ref[...], qc_ref[...]], axis=1)
    qq128 = jnp.concatenate([q, q], axis=1)
    tq = jnp.dot(q, wq_ref[...], preferred_element_type=jnp.float32,
                 precision=PREC) + b1_ref[...]
    kp = kp_ref[...]
    kp3 = kp.reshape(BB, H, 4 * D)
    qkp = (kp3 * qq128[:, None, :]).reshape(BB * H, 4 * D)
    h1 = (jnp.dot(kp, wbk_ref[...], preferred_element_type=jnp.float32,
                  precision=PREC)
          + jnp.dot(qkp, wbp_ref[...], preferred_element_type=jnp.float32,
                    precision=PREC))
    h1 = h1.reshape(BB, H, 160) + tq[:, None, :]
    h1 = jnp.where(h1 > 0, h1, a1 * h1).reshape(BB * H, 160)
    h2 = jnp.dot(h1, w2b_ref[...], preferred_element_type=jnp.float32,
                 precision=PREC) + b2_ref[...]
    h2 = jnp.where(h2 > 0, h2, a2 * h2)
    h23 = h2.reshape(BB, H, 80)
    s_e = (jnp.sum(h23 * woe_ref[...][None, :, :], axis=2) + bo) * jnp.float32(0.125)
    s_o = (jnp.sum(h23 * woo_ref[...][None, :, :], axis=2) + bo) * jnp.float32(0.125)
    lens = len_ref[0, 0, :]
    jidx = lax.broadcasted_iota(jnp.int32, (BB, H), 1)
    neg = jnp.float32(-1e9)
    s_e = jnp.where(2 * jidx < lens[:, None], s_e, neg)
    s_o = jnp.where(2 * jidx + 1 < lens[:, None], s_o, neg)
    mx = jnp.maximum(jnp.max(s_e, axis=1, keepdims=True),
                     jnp.max(s_o, axis=1, keepdims=True))
    e_e = jnp.exp(s_e - mx)
    e_o = jnp.exp(s_o - mx)
    den = jnp.sum(e_e, axis=1, keepdims=True) + jnp.sum(e_o, axis=1, keepdims=True)
    a_e = e_e / den
    a_o = e_o / den
    lane = lax.broadcasted_iota(jnp.int32, (BB, H, 4 * D), 2)
    att128 = jnp.where(lane < 2 * D, a_e[:, :, None], a_o[:, :, None])
    pooled = jnp.sum(kp3 * att128, axis=1)
    out_ref[...] = pooled[:, 0:2 * D] + pooled[:, 2 * D:4 * D]


def _attention(q_item, q_cate, kp, len3, wq, b1_2, wbk, wbp, w2b, b2_2,
               wo_e, wo_o, sc):
    rep = lambda shape: pl.BlockSpec(shape, lambda i: tuple(0 for _ in shape))
    return pl.pallas_call(
        _att_body,
        grid=(B // BB,),
        in_specs=[
            pl.BlockSpec((BB, D), lambda i: (i, 0)),
            pl.BlockSpec((BB, D), lambda i: (i, 0)),
            pl.BlockSpec((BB * H, 4 * D), lambda i: (i, 0)),
            pl.BlockSpec((1, 1, BB), lambda i: (i, 0, 0)),
            rep((2 * D, 160)),
            rep((1, 160)),
            rep((4 * D, 160)),
            rep((4 * D, 160)),
            rep((160, 80)),
            rep((1, 80)),
            rep((1, 80)),
            rep((1, 80)),
            rep((1, 8)),
        ],
        out_specs=pl.BlockSpec((BB, 2 * D), lambda i: (i, 0)),
        out_shape=jax.ShapeDtypeStruct((B, 2 * D), jnp.float32),
    )(q_item, q_cate, kp, len3, wq, b1_2, wbk, wbp, w2b, b2_2, wo_e, wo_o, sc)


def _mlp_body(pr_ref, qi_ref, qc_ref, ao_ref, w1_ref, b1_ref, g1_ref, be1_ref,
              w2_ref, b2_ref, g2_ref, be2_ref, fw_ref, sc_ref, out_ref):
    a1 = sc_ref[0, 0]
    a2 = sc_ref[0, 1]
    fb = sc_ref[0, 2]
    x = jnp.concatenate([pr_ref[...], qi_ref[...], qc_ref[...], ao_ref[...]],
                        axis=1)
    h = jnp.dot(x, w1_ref[...], preferred_element_type=jnp.float32,
                precision=PREC) + b1_ref[...]
    mu = jnp.mean(h, axis=0, keepdims=True)
    var = jnp.mean((h - mu) ** 2, axis=0, keepdims=True)
    h = g1_ref[...] * (h - mu) / jnp.sqrt(var + 1e-5) + be1_ref[...]
    h = jnp.where(h > 0, h, a1 * h)
    h = jnp.dot(h, w2_ref[...], preferred_element_type=jnp.float32,
                precision=PREC) + b2_ref[...]
    mu = jnp.mean(h, axis=0, keepdims=True)
    var = jnp.mean((h - mu) ** 2, axis=0, keepdims=True)
    h = g2_ref[...] * (h - mu) / jnp.sqrt(var + 1e-5) + be2_ref[...]
    h = jnp.where(h > 0, h, a2 * h)
    o = jnp.dot(h, fw_ref[...], preferred_element_type=jnp.float32,
                precision=PREC) + fb
    out_ref[...] = jax.nn.sigmoid(o)


def _mlp(price2, q_item, q_cate, att_out,
         w1, b1, g1, be1, w2, b2, g2, be2, fw, sc):
    return pl.pallas_call(
        _mlp_body,
        out_shape=jax.ShapeDtypeStruct((B, 1), jnp.float32),
    )(price2, q_item, q_cate, att_out, w1, b1, g1, be1, w2, b2, g2, be2,
      fw, sc)


def kernel(price, item_id, cate_id, hist_item_id, hist_cate_id,
           hist_item_id_length, hist_cate_id_length,
           emb_item, emb_cate, emb_hist_item, emb_hist_cate,
           att_W1, att_b1, att_a1, att_W2, att_b2, att_a2, att_Wo, att_bo,
           mlp_W1, mlp_b1, mlp_g1, mlp_be1, mlp_a1,
           mlp_W2, mlp_b2, mlp_g2, mlp_be2, mlp_a2,
           fin_W, fin_b):
    kp, q_item, q_cate = _sc_gather_all(
        emb_item, emb_cate, emb_hist_item, emb_hist_cate,
        item_id, cate_id,
        _parity_perm(hist_item_id), _parity_perm(hist_cate_id))

    wq = att_W1[0:2 * D] + att_W1[4 * D:6 * D]
    wk = att_W1[2 * D:4 * D] - att_W1[4 * D:6 * D]
    wp = att_W1[6 * D:8 * D]
    z64 = jnp.zeros((2 * D, 80), jnp.float32)
    z80 = jnp.zeros((80, 40), jnp.float32)
    wq2 = jnp.concatenate([wq, wq], axis=1)
    wbk = jnp.block([[wk, z64], [z64, wk]])
    wbp = jnp.block([[wp, z64], [z64, wp]])
    w2b = jnp.block([[att_W2, z80], [z80, att_W2]])
    b1_2 = jnp.concatenate([att_b1, att_b1]).reshape(1, 160)
    b2_2 = jnp.concatenate([att_b2, att_b2]).reshape(1, 80)
    wo = att_Wo.reshape(40)
    z40 = jnp.zeros((40,), jnp.float32)
    wo_e = jnp.concatenate([wo, z40]).reshape(1, 80)
    wo_o = jnp.concatenate([z40, wo]).reshape(1, 80)
    keys_len = jnp.minimum(hist_item_id_length, hist_cate_id_length)
    len3 = keys_len.astype(jnp.int32).reshape(B // BB, 1, BB)
    sc_att = jnp.stack([att_a1, att_a2, att_bo[0]] + [jnp.float32(0)] * 5)
    att_out = _attention(q_item, q_cate, kp, len3, wq2, b1_2,
                         wbk, wbp, w2b, b2_2, wo_e, wo_o,
                         sc_att.reshape(1, 8))

    sc_mlp = jnp.stack([mlp_a1, mlp_a2, fin_b[0]] + [jnp.float32(0)] * 5)
    return _mlp(price.reshape(B, 1), q_item, q_cate, att_out,
                mlp_W1, mlp_b1.reshape(1, 200), mlp_g1.reshape(1, 200),
                mlp_be1.reshape(1, 200), mlp_W2, mlp_b2.reshape(1, 80),
                mlp_g2.reshape(1, 80), mlp_be2.reshape(1, 80),
                fin_W, sc_mlp.reshape(1, 8))


# speedup vs baseline: 3.8522x; 1.2115x over previous
import functools

import jax
import jax.numpy as jnp
from jax import lax
from jax.experimental import pallas as pl
from jax.experimental.pallas import tpu as pltpu
from jax.experimental.pallas import tpu_sc as plsc

B = 4096
T = 200
D = 32
N = B * T
NH = N // 2
NW = 32
CH = 800
BB = 64
NB2 = B // BB // 2

PREC = jax.lax.Precision.DEFAULT


def _sc_gather_all(emb_item, emb_cate, emb_hist_item, emb_hist_cate,
                   item_id, cate_id, hist_item_flat, hist_cate_flat):
    per_w = NH // NW
    n_ch = per_w // CH
    qn = B // NW

    mesh = plsc.VectorSubcoreMesh(core_axis_name="c", subcore_axis_name="s")

    @functools.partial(
        pl.kernel,
        out_type=[jax.ShapeDtypeStruct((NH, 4 * D), jnp.float32),
                  jax.ShapeDtypeStruct((B, D), jnp.float32),
                  jax.ShapeDtypeStruct((B, D), jnp.float32)],
        mesh=mesh,
        scratch_types=[pltpu.VMEM((CH,), jnp.int32),
                       pltpu.VMEM((CH, D), jnp.float32),
                       pltpu.VMEM((qn,), jnp.int32),
                       pltpu.VMEM((qn, D), jnp.float32),
                       pltpu.SemaphoreType.DMA],
        compiler_params=pltpu.CompilerParams(use_tc_tiling_on_sc=False),
    )
    def gather_kernel(ei_hbm, ec_hbm, ehi_hbm, ehc_hbm,
                      ii_hbm, ci_hbm, hi_hbm, hc_hbm,
                      kp_out, qi_out, qc_out,
                      idx_v, rows_v, qidx_v, qrows_v, sem):
        wid = lax.axis_index("s") * 2 + lax.axis_index("c")
        base = wid * per_w

        @pl.loop(0, n_ch)
        def _(c):
            off = base + c * CH
            pltpu.sync_copy(hi_hbm.at[pl.ds(off, CH)], idx_v)
            pltpu.async_copy(ehi_hbm.at[idx_v], rows_v, sem).wait()
            pltpu.sync_copy(rows_v, kp_out.at[pl.ds(off, CH), pl.ds(0, D)])
            pltpu.sync_copy(hc_hbm.at[pl.ds(off, CH)], idx_v)
            pltpu.async_copy(ehc_hbm.at[idx_v], rows_v, sem).wait()
            pltpu.sync_copy(rows_v, kp_out.at[pl.ds(off, CH), pl.ds(D, D)])
            pltpu.sync_copy(hi_hbm.at[pl.ds(NH + off, CH)], idx_v)
            pltpu.async_copy(ehi_hbm.at[idx_v], rows_v, sem).wait()
            pltpu.sync_copy(rows_v, kp_out.at[pl.ds(off, CH), pl.ds(2 * D, D)])
            pltpu.sync_copy(hc_hbm.at[pl.ds(NH + off, CH)], idx_v)
            pltpu.async_copy(ehc_hbm.at[idx_v], rows_v, sem).wait()
            pltpu.sync_copy(rows_v, kp_out.at[pl.ds(off, CH), pl.ds(3 * D, D)])

        qbase = wid * qn
        pltpu.sync_copy(ii_hbm.at[pl.ds(qbase, qn)], qidx_v)
        pltpu.async_copy(ei_hbm.at[qidx_v], qrows_v, sem).wait()
        pltpu.sync_copy(qrows_v, qi_out.at[pl.ds(qbase, qn)])
        pltpu.sync_copy(ci_hbm.at[pl.ds(qbase, qn)], qidx_v)
        pltpu.async_copy(ec_hbm.at[qidx_v], qrows_v, sem).wait()
        pltpu.sync_copy(qrows_v, qc_out.at[pl.ds(qbase, qn)])

    return gather_kernel(emb_item, emb_cate, emb_hist_item, emb_hist_cate,
                         item_id, cate_id, hist_item_flat, hist_cate_flat)


def _att_body(qil_ref, qcl_ref, qih_ref, qch_ref, kp_ref, lenl_ref, lenh_ref,
              wbq_ref, b1_ref, wbk_ref, wbp_ref, w2b_ref, b2_ref,
              wol_ref, woh_ref, sc_ref, out_ref):
    a1 = sc_ref[0, 0]
    a2 = sc_ref[0, 1]
    bo = sc_ref[0, 2]
    qq128 = jnp.concatenate([qil_ref[...], qcl_ref[...],
                             qih_ref[...], qch_ref[...]], axis=1)
    tq = jnp.dot(qq128, wbq_ref[...], preferred_element_type=jnp.float32,
                 precision=PREC) + b1_ref[...]
    kp = kp_ref[...]
    kp3 = kp.reshape(BB, T, 4 * D)
    qkp = (kp3 * qq128[:, None, :]).reshape(BB * T, 4 * D)
    h1 = (jnp.dot(kp, wbk_ref[...], preferred_element_type=jnp.float32,
                  precision=PREC)
          + jnp.dot(qkp, wbp_ref[...], preferred_element_type=jnp.float32,
                    precision=PREC))
    h1 = h1.reshape(BB, T, 160) + tq[:, None, :]
    h1 = jnp.where(h1 > 0, h1, a1 * h1).reshape(BB * T, 160)
    h2 = jnp.dot(h1, w2b_ref[...], preferred_element_type=jnp.float32,
                 precision=PREC) + b2_ref[...]
    h2 = jnp.where(h2 > 0, h2, a2 * h2)
    h23 = h2.reshape(BB, T, 80)
    s_lo = (jnp.sum(h23 * wol_ref[...][None, :, :], axis=2) + bo) * jnp.float32(0.125)
    s_hi = (jnp.sum(h23 * woh_ref[...][None, :, :], axis=2) + bo) * jnp.float32(0.125)
    lens_lo = lenl_ref[0, 0, :]
    lens_hi = lenh_ref[0, 0, :]
    jidx = lax.broadcasted_iota(jnp.int32, (BB, T), 1)
    neg = jnp.float32(-1e9)
    s_lo = jnp.where(jidx < lens_lo[:, None], s_lo, neg)
    s_hi = jnp.where(jidx < lens_hi[:, None], s_hi, neg)
    e_lo = jnp.exp(s_lo - jnp.max(s_lo, axis=1, keepdims=True))
    e_hi = jnp.exp(s_hi - jnp.max(s_hi, axis=1, keepdims=True))
    a_lo = e_lo / jnp.sum(e_lo, axis=1, keepdims=True)
    a_hi = e_hi / jnp.sum(e_hi, axis=1, keepdims=True)
    lane = lax.broadcasted_iota(jnp.int32, (BB, T, 4 * D), 2)
    att128 = jnp.where(lane < 2 * D, a_lo[:, :, None], a_hi[:, :, None])
    out_ref[...] = jnp.sum(kp3 * att128, axis=1)


def _attention(q_item, q_cate, kp, len3, wbq, b1_2, wbk, wbp, w2b, b2_2,
               wo_lo, wo_hi, sc):
    rep = lambda shape: pl.BlockSpec(shape, lambda i: tuple(0 for _ in shape))
    return pl.pallas_call(
        _att_body,
        grid=(NB2,),
        in_specs=[
            pl.BlockSpec((BB, D), lambda i: (i, 0)),
            pl.BlockSpec((BB, D), lambda i: (i, 0)),
            pl.BlockSpec((BB, D), lambda i: (i + NB2, 0)),
            pl.BlockSpec((BB, D), lambda i: (i + NB2, 0)),
            pl.BlockSpec((BB * T, 4 * D), lambda i: (i, 0)),
            pl.BlockSpec((1, 1, BB), lambda i: (i, 0, 0)),
            pl.BlockSpec((1, 1, BB), lambda i: (i + NB2, 0, 0)),
            rep((4 * D, 160)),
            rep((1, 160)),
            rep((4 * D, 160)),
            rep((4 * D, 160)),
            rep((160, 80)),
            rep((1, 80)),
            rep((1, 80)),
            rep((1, 80)),
            rep((1, 8)),
        ],
        out_specs=pl.BlockSpec((BB, 4 * D), lambda i: (i, 0)),
        out_shape=jax.ShapeDtypeStruct((B // 2, 4 * D), jnp.float32),
    )(q_item, q_cate, q_item, q_cate, kp, len3, len3,
      wbq, b1_2, wbk, wbp, w2b, b2_2, wo_lo, wo_hi, sc)


def _mlp_body(pr_ref, qi_ref, qc_ref, ao_ref, w1_ref, b1_ref, g1_ref, be1_ref,
              w2_ref, b2_ref, g2_ref, be2_ref, fw_ref, sc_ref, out_ref):
    a1 = sc_ref[0, 0]
    a2 = sc_ref[0, 1]
    fb = sc_ref[0, 2]
    ao = ao_ref[...]
    att = jnp.concatenate([ao[:, 0:2 * D], ao[:, 2 * D:4 * D]], axis=0)
    x = jnp.concatenate([pr_ref[...], qi_ref[...], qc_ref[...], att],
                        axis=1)
    h = jnp.dot(x, w1_ref[...], preferred_element_type=jnp.float32,
                precision=PREC) + b1_ref[...]
    mu = jnp.mean(h, axis=0, keepdims=True)
    var = jnp.mean((h - mu) ** 2, axis=0, keepdims=True)
    h = g1_ref[...] * (h - mu) / jnp.sqrt(var + 1e-5) + be1_ref[...]
    h = jnp.where(h > 0, h, a1 * h)
    h = jnp.dot(h, w2_ref[...], preferred_element_type=jnp.float32,
                precision=PREC) + b2_ref[...]
    mu = jnp.mean(h, axis=0, keepdims=True)
    var = jnp.mean((h - mu) ** 2, axis=0, keepdims=True)
    h = g2_ref[...] * (h - mu) / jnp.sqrt(var + 1e-5) + be2_ref[...]
    h = jnp.where(h > 0, h, a2 * h)
    o = jnp.dot(h, fw_ref[...], preferred_element_type=jnp.float32,
                precision=PREC) + fb
    out_ref[...] = jax.nn.sigmoid(o)


def _mlp(price2, q_item, q_cate, att_fold,
         w1, b1, g1, be1, w2, b2, g2, be2, fw, sc):
    return pl.pallas_call(
        _mlp_body,
        out_shape=jax.ShapeDtypeStruct((B, 1), jnp.float32),
    )(price2, q_item, q_cate, att_fold, w1, b1, g1, be1, w2, b2, g2, be2,
      fw, sc)


def kernel(price, item_id, cate_id, hist_item_id, hist_cate_id,
           hist_item_id_length, hist_cate_id_length,
           emb_item, emb_cate, emb_hist_item, emb_hist_cate,
           att_W1, att_b1, att_a1, att_W2, att_b2, att_a2, att_Wo, att_bo,
           mlp_W1, mlp_b1, mlp_g1, mlp_be1, mlp_a1,
           mlp_W2, mlp_b2, mlp_g2, mlp_be2, mlp_a2,
           fin_W, fin_b):
    kp, q_item, q_cate = _sc_gather_all(
        emb_item, emb_cate, emb_hist_item, emb_hist_cate,
        item_id, cate_id,
        hist_item_id.reshape(-1), hist_cate_id.reshape(-1))

    wq = att_W1[0:2 * D] + att_W1[4 * D:6 * D]
    wk = att_W1[2 * D:4 * D] - att_W1[4 * D:6 * D]
    wp = att_W1[6 * D:8 * D]
    z64 = jnp.zeros((2 * D, 80), jnp.float32)
    z80 = jnp.zeros((80, 40), jnp.float32)
    wbq = jnp.block([[wq, z64], [z64, wq]])
    wbk = jnp.block([[wk, z64], [z64, wk]])
    wbp = jnp.block([[wp, z64], [z64, wp]])
    w2b = jnp.block([[att_W2, z80], [z80, att_W2]])
    b1_2 = jnp.concatenate([att_b1, att_b1]).reshape(1, 160)
    b2_2 = jnp.concatenate([att_b2, att_b2]).reshape(1, 80)
    wo = att_Wo.reshape(40)
    z40 = jnp.zeros((40,), jnp.float32)
    wo_lo = jnp.concatenate([wo, z40]).reshape(1, 80)
    wo_hi = jnp.concatenate([z40, wo]).reshape(1, 80)
    keys_len = jnp.minimum(hist_item_id_length, hist_cate_id_length)
    len3 = keys_len.astype(jnp.int32).reshape(B // BB, 1, BB)
    sc_att = jnp.stack([att_a1, att_a2, att_bo[0]] + [jnp.float32(0)] * 5)
    att_fold = _attention(q_item, q_cate, kp, len3, wbq, b1_2,
                          wbk, wbp, w2b, b2_2, wo_lo, wo_hi,
                          sc_att.reshape(1, 8))

    sc_mlp = jnp.stack([mlp_a1, mlp_a2, fin_b[0]] + [jnp.float32(0)] * 5)
    return _mlp(price.reshape(B, 1), q_item, q_cate, att_fold,
                mlp_W1, mlp_b1.reshape(1, 200), mlp_g1.reshape(1, 200),
                mlp_be1.reshape(1, 200), mlp_W2, mlp_b2.reshape(1, 80),
                mlp_g2.reshape(1, 80), mlp_be2.reshape(1, 80),
                fin_W, sc_mlp.reshape(1, 8))


# speedup vs baseline: 4.3549x; 1.1305x over previous
import functools

import jax
import jax.numpy as jnp
from jax import lax
from jax.experimental import pallas as pl
from jax.experimental.pallas import tpu as pltpu
from jax.experimental.pallas import tpu_sc as plsc

B = 4096
T = 200
D = 32
N = B * T
NH = N // 2
NW = 32
CH = 800
BB = 64
NB2 = B // BB // 2

PREC = jax.lax.Precision.DEFAULT


def _sc_gather_q(emb_item, emb_cate, item_id, cate_id):
    nsc = 2
    qn = B // nsc
    mesh = plsc.ScalarSubcoreMesh(axis_name="c", num_cores=nsc)

    @functools.partial(
        pl.kernel,
        out_type=[jax.ShapeDtypeStruct((B, D), jnp.float32),
                  jax.ShapeDtypeStruct((B, D), jnp.float32)],
        mesh=mesh,
        scratch_types=[pltpu.SMEM((qn,), jnp.int32),
                       pltpu.SemaphoreType.DMA,
                       pltpu.SemaphoreType.DMA],
    )
    def q_kernel(ei_hbm, ec_hbm, ii_hbm, ci_hbm, qi_out, qc_out,
                 idx_s, gsem, ssem):
        qbase = lax.axis_index("c") * qn

        def one_table(table, idx_hbm, out):
            pltpu.async_copy(idx_hbm.at[pl.ds(qbase, qn)], idx_s, ssem).wait()

            @pl.loop(0, qn)
            def _(j):
                pltpu.async_copy(table.at[pl.ds(idx_s[j], 1)],
                                 out.at[pl.ds(qbase + j, 1)], gsem)

            pltpu.make_async_copy(table.at[pl.ds(0, qn)],
                                  out.at[pl.ds(qbase, qn)], gsem).wait()

        one_table(ei_hbm, ii_hbm, qi_out)
        one_table(ec_hbm, ci_hbm, qc_out)

    return q_kernel(emb_item, emb_cate, item_id, cate_id)


def _sc_gather_hist(emb_hist_item, emb_hist_cate,
                    hist_item_flat, hist_cate_flat):
    per_w = NH // NW
    n_ch = per_w // CH

    mesh = plsc.VectorSubcoreMesh(core_axis_name="c", subcore_axis_name="s")

    @functools.partial(
        pl.kernel,
        out_type=jax.ShapeDtypeStruct((NH, 4 * D), jnp.float32),
        mesh=mesh,
        scratch_types=[[pltpu.VMEM((CH,), jnp.int32) for _ in range(4)],
                       [pltpu.VMEM((CH, D), jnp.float32) for _ in range(4)],
                       pltpu.SemaphoreType.DMA,
                       pltpu.SemaphoreType.DMA],
        compiler_params=pltpu.CompilerParams(use_tc_tiling_on_sc=False),
    )
    def gather_kernel(ehi_hbm, ehc_hbm, hi_hbm, hc_hbm, kp_out,
                      idx_vs, rows_vs, gsem, isem):
        wid = lax.axis_index("s") * 2 + lax.axis_index("c")
        base = wid * per_w

        @pl.loop(0, n_ch)
        def _(c):
            off = base + c * CH
            srcs = ((hi_hbm, ehi_hbm, 0), (hc_hbm, ehc_hbm, 0),
                    (hi_hbm, ehi_hbm, NH), (hc_hbm, ehc_hbm, NH))
            hs = [pltpu.async_copy(idx.at[pl.ds(ioff + off, CH)],
                                   idx_vs[k], isem)
                  for k, (idx, _, ioff) in enumerate(srcs)]
            for h in hs:
                h.wait()
            hs = [pltpu.async_copy(tab.at[idx_vs[k]], rows_vs[k], gsem)
                  for k, (_, tab, _) in enumerate(srcs)]
            for h in hs:
                h.wait()
            hs = [pltpu.async_copy(rows_vs[k],
                                   kp_out.at[pl.ds(off, CH), pl.ds(k * D, D)],
                                   isem)
                  for k in range(4)]
            for h in hs:
                h.wait()

    return gather_kernel(emb_hist_item, emb_hist_cate,
                         hist_item_flat, hist_cate_flat)


def _att_body(qil_ref, qcl_ref, qih_ref, qch_ref, kp_ref, lenl_ref, lenh_ref,
              wbq_ref, b1_ref, wbk_ref, wbp_ref, w2b_ref, b2_ref,
              wol_ref, woh_ref, sc_ref, out_ref):
    a1 = sc_ref[0, 0]
    a2 = sc_ref[0, 1]
    bo = sc_ref[0, 2]
    qq128 = jnp.concatenate([qil_ref[...], qcl_ref[...],
                             qih_ref[...], qch_ref[...]], axis=1)
    tq = jnp.dot(qq128, wbq_ref[...], preferred_element_type=jnp.float32,
                 precision=PREC) + b1_ref[...]
    kp = kp_ref[...]
    kp3 = kp.reshape(BB, T, 4 * D)
    qkp = (kp3 * qq128[:, None, :]).reshape(BB * T, 4 * D)
    h1 = (jnp.dot(kp, wbk_ref[...], preferred_element_type=jnp.float32,
                  precision=PREC)
          + jnp.dot(qkp, wbp_ref[...], preferred_element_type=jnp.float32,
                    precision=PREC))
    h1 = h1.reshape(BB, T, 160) + tq[:, None, :]
    h1 = jnp.where(h1 > 0, h1, a1 * h1).reshape(BB * T, 160)
    h2 = jnp.dot(h1, w2b_ref[...], preferred_element_type=jnp.float32,
                 precision=PREC) + b2_ref[...]
    h2 = jnp.where(h2 > 0, h2, a2 * h2)
    h23 = h2.reshape(BB, T, 80)
    s_lo = (jnp.sum(h23 * wol_ref[...][None, :, :], axis=2) + bo) * jnp.float32(0.125)
    s_hi = (jnp.sum(h23 * woh_ref[...][None, :, :], axis=2) + bo) * jnp.float32(0.125)
    lens_lo = lenl_ref[0, 0, :]
    lens_hi = lenh_ref[0, 0, :]
    jidx = lax.broadcasted_iota(jnp.int32, (BB, T), 1)
    neg = jnp.float32(-1e9)
    s_lo = jnp.where(jidx < lens_lo[:, None], s_lo, neg)
    s_hi = jnp.where(jidx < lens_hi[:, None], s_hi, neg)
    e_lo = jnp.exp(s_lo - jnp.max(s_lo, axis=1, keepdims=True))
    e_hi = jnp.exp(s_hi - jnp.max(s_hi, axis=1, keepdims=True))
    a_lo = e_lo / jnp.sum(e_lo, axis=1, keepdims=True)
    a_hi = e_hi / jnp.sum(e_hi, axis=1, keepdims=True)
    lane = lax.broadcasted_iota(jnp.int32, (BB, T, 4 * D), 2)
    att128 = jnp.where(lane < 2 * D, a_lo[:, :, None], a_hi[:, :, None])
    out_ref[...] = jnp.sum(kp3 * att128, axis=1)


def _attention(q_item, q_cate, kp, len3, wbq, b1_2, wbk, wbp, w2b, b2_2,
               wo_lo, wo_hi, sc):
    rep = lambda shape: pl.BlockSpec(shape, lambda i: tuple(0 for _ in shape))
    return pl.pallas_call(
        _att_body,
        grid=(NB2,),
        in_specs=[
            pl.BlockSpec((BB, D), lambda i: (i, 0)),
            pl.BlockSpec((BB, D), lambda i: (i, 0)),
            pl.BlockSpec((BB, D), lambda i: (i + NB2, 0)),
            pl.BlockSpec((BB, D), lambda i: (i + NB2, 0)),
            pl.BlockSpec((BB * T, 4 * D), lambda i: (i, 0)),
            pl.BlockSpec((1, 1, BB), lambda i: (i, 0, 0)),
            pl.BlockSpec((1, 1, BB), lambda i: (i + NB2, 0, 0)),
            rep((4 * D, 160)),
            rep((1, 160)),
            rep((4 * D, 160)),
            rep((4 * D, 160)),
            rep((160, 80)),
            rep((1, 80)),
            rep((1, 80)),
            rep((1, 80)),
            rep((1, 8)),
        ],
        out_specs=pl.BlockSpec((BB, 4 * D), lambda i: (i, 0)),
        out_shape=jax.ShapeDtypeStruct((B // 2, 4 * D), jnp.float32),
    )(q_item, q_cate, q_item, q_cate, kp, len3, len3,
      wbq, b1_2, wbk, wbp, w2b, b2_2, wo_lo, wo_hi, sc)


def _mlp_body(pr_ref, qi_ref, qc_ref, ao_ref, w1_ref, b1_ref, g1_ref, be1_ref,
              w2_ref, b2_ref, g2_ref, be2_ref, fw_ref, sc_ref, out_ref):
    a1 = sc_ref[0, 0]
    a2 = sc_ref[0, 1]
    fb = sc_ref[0, 2]
    ao = ao_ref[...]
    att = jnp.concatenate([ao[:, 0:2 * D], ao[:, 2 * D:4 * D]], axis=0)
    x = jnp.concatenate([pr_ref[...], qi_ref[...], qc_ref[...], att],
                        axis=1)
    h = jnp.dot(x, w1_ref[...], preferred_element_type=jnp.float32,
                precision=PREC) + b1_ref[...]
    mu = jnp.mean(h, axis=0, keepdims=True)
    var = jnp.mean((h - mu) ** 2, axis=0, keepdims=True)
    h = g1_ref[...] * (h - mu) / jnp.sqrt(var + 1e-5) + be1_ref[...]
    h = jnp.where(h > 0, h, a1 * h)
    h = jnp.dot(h, w2_ref[...], preferred_element_type=jnp.float32,
                precision=PREC) + b2_ref[...]
    mu = jnp.mean(h, axis=0, keepdims=True)
    var = jnp.mean((h - mu) ** 2, axis=0, keepdims=True)
    h = g2_ref[...] * (h - mu) / jnp.sqrt(var + 1e-5) + be2_ref[...]
    h = jnp.where(h > 0, h, a2 * h)
    o = jnp.dot(h, fw_ref[...], preferred_element_type=jnp.float32,
                precision=PREC) + fb
    out_ref[...] = jax.nn.sigmoid(o)


def _mlp(price2, q_item, q_cate, att_fold,
         w1, b1, g1, be1, w2, b2, g2, be2, fw, sc):
    return pl.pallas_call(
        _mlp_body,
        out_shape=jax.ShapeDtypeStruct((B, 1), jnp.float32),
    )(price2, q_item, q_cate, att_fold, w1, b1, g1, be1, w2, b2, g2, be2,
      fw, sc)


def kernel(price, item_id, cate_id, hist_item_id, hist_cate_id,
           hist_item_id_length, hist_cate_id_length,
           emb_item, emb_cate, emb_hist_item, emb_hist_cate,
           att_W1, att_b1, att_a1, att_W2, att_b2, att_a2, att_Wo, att_bo,
           mlp_W1, mlp_b1, mlp_g1, mlp_be1, mlp_a1,
           mlp_W2, mlp_b2, mlp_g2, mlp_be2, mlp_a2,
           fin_W, fin_b):
    q_item, q_cate = _sc_gather_q(emb_item, emb_cate, item_id, cate_id)
    kp = _sc_gather_hist(emb_hist_item, emb_hist_cate,
                         hist_item_id.reshape(-1), hist_cate_id.reshape(-1))

    wq = att_W1[0:2 * D] + att_W1[4 * D:6 * D]
    wk = att_W1[2 * D:4 * D] - att_W1[4 * D:6 * D]
    wp = att_W1[6 * D:8 * D]
    z64 = jnp.zeros((2 * D, 80), jnp.float32)
    z80 = jnp.zeros((80, 40), jnp.float32)
    wbq = jnp.block([[wq, z64], [z64, wq]])
    wbk = jnp.block([[wk, z64], [z64, wk]])
    wbp = jnp.block([[wp, z64], [z64, wp]])
    w2b = jnp.block([[att_W2, z80], [z80, att_W2]])
    b1_2 = jnp.concatenate([att_b1, att_b1]).reshape(1, 160)
    b2_2 = jnp.concatenate([att_b2, att_b2]).reshape(1, 80)
    wo = att_Wo.reshape(40)
    z40 = jnp.zeros((40,), jnp.float32)
    wo_lo = jnp.concatenate([wo, z40]).reshape(1, 80)
    wo_hi = jnp.concatenate([z40, wo]).reshape(1, 80)
    keys_len = jnp.minimum(hist_item_id_length, hist_cate_id_length)
    len3 = keys_len.astype(jnp.int32).reshape(B // BB, 1, BB)
    sc_att = jnp.stack([att_a1, att_a2, att_bo[0]] + [jnp.float32(0)] * 5)
    att_fold = _attention(q_item, q_cate, kp, len3, wbq, b1_2,
                          wbk, wbp, w2b, b2_2, wo_lo, wo_hi,
                          sc_att.reshape(1, 8))

    sc_mlp = jnp.stack([mlp_a1, mlp_a2, fin_b[0]] + [jnp.float32(0)] * 5)
    return _mlp(price.reshape(B, 1), q_item, q_cate, att_fold,
                mlp_W1, mlp_b1.reshape(1, 200), mlp_g1.reshape(1, 200),
                mlp_be1.reshape(1, 200), mlp_W2, mlp_b2.reshape(1, 80),
                mlp_g2.reshape(1, 80), mlp_be2.reshape(1, 80),
                fin_W, sc_mlp.reshape(1, 8))


# speedup vs baseline: 4.3633x; 1.0019x over previous
import functools

import jax
import jax.numpy as jnp
from jax import lax
from jax.experimental import pallas as pl
from jax.experimental.pallas import tpu as pltpu
from jax.experimental.pallas import tpu_sc as plsc

B = 4096
T = 200
D = 32
N = B * T
NH = N // 2
NW = 32
CH = 800
BB = 64
NB2 = B // BB // 2

PREC = jax.lax.Precision.DEFAULT


def _sc_gather_q(emb_item, emb_cate, item_id, cate_id):
    nsc = 2
    qn = B // nsc
    mesh = plsc.ScalarSubcoreMesh(axis_name="c", num_cores=nsc)

    @functools.partial(
        pl.kernel,
        out_type=[jax.ShapeDtypeStruct((B, D), jnp.float32),
                  jax.ShapeDtypeStruct((B, D), jnp.float32)],
        mesh=mesh,
        scratch_types=[pltpu.SMEM((qn,), jnp.int32),
                       pltpu.SemaphoreType.DMA,
                       pltpu.SemaphoreType.DMA],
        compiler_params=pltpu.CompilerParams(use_tc_tiling_on_sc=True),
    )
    def q_kernel(ei_hbm, ec_hbm, ii_hbm, ci_hbm, qi_out, qc_out,
                 idx_s, gsem, ssem):
        qbase = lax.axis_index("c") * qn

        def one_table(table, idx_hbm, out):
            pltpu.async_copy(idx_hbm.at[pl.ds(qbase, qn)], idx_s, ssem).wait()

            @pl.loop(0, qn)
            def _(j):
                pltpu.async_copy(table.at[pl.ds(idx_s[j], 1)],
                                 out.at[pl.ds(qbase + j, 1)], gsem)

            pltpu.make_async_copy(table.at[pl.ds(0, qn)],
                                  out.at[pl.ds(qbase, qn)], gsem).wait()

        one_table(ei_hbm, ii_hbm, qi_out)
        one_table(ec_hbm, ci_hbm, qc_out)

    return q_kernel(emb_item, emb_cate, item_id, cate_id)


def _sc_gather_hist(emb_hist_item, emb_hist_cate,
                    hist_item_flat, hist_cate_flat):
    per_w = NH // NW
    n_ch = per_w // CH

    mesh = plsc.VectorSubcoreMesh(core_axis_name="c", subcore_axis_name="s")

    @functools.partial(
        pl.kernel,
        out_type=jax.ShapeDtypeStruct((NH, 4 * D), jnp.float32),
        mesh=mesh,
        scratch_types=[[pltpu.VMEM((CH,), jnp.int32) for _ in range(4)],
                       [pltpu.VMEM((CH, D), jnp.float32) for _ in range(4)],
                       pltpu.SemaphoreType.DMA,
                       pltpu.SemaphoreType.DMA],
        compiler_params=pltpu.CompilerParams(use_tc_tiling_on_sc=False),
    )
    def gather_kernel(ehi_hbm, ehc_hbm, hi_hbm, hc_hbm, kp_out,
                      idx_vs, rows_vs, gsem, isem):
        wid = lax.axis_index("s") * 2 + lax.axis_index("c")
        base = wid * per_w

        @pl.loop(0, n_ch)
        def _(c):
            off = base + c * CH
            srcs = ((hi_hbm, ehi_hbm, 0), (hc_hbm, ehc_hbm, 0),
                    (hi_hbm, ehi_hbm, NH), (hc_hbm, ehc_hbm, NH))
            hs = [pltpu.async_copy(idx.at[pl.ds(ioff + off, CH)],
                                   idx_vs[k], isem)
                  for k, (idx, _, ioff) in enumerate(srcs)]
            for h in hs:
                h.wait()
            ws = []
            for k, (_, tab, _) in enumerate(srcs):
                pltpu.async_copy(tab.at[idx_vs[k]], rows_vs[k], gsem).wait()
                ws.append(pltpu.async_copy(
                    rows_vs[k],
                    kp_out.at[pl.ds(off, CH), pl.ds(k * D, D)], isem))
            for h in ws:
                h.wait()

    return gather_kernel(emb_hist_item, emb_hist_cate,
                         hist_item_flat, hist_cate_flat)


def _att_body(qil_ref, qcl_ref, qih_ref, qch_ref, kp_ref, lenl_ref, lenh_ref,
              wbq_ref, b1_ref, wbk_ref, wbp_ref, w2b_ref, b2_ref,
              wol_ref, woh_ref, sc_ref, out_ref):
    a1 = sc_ref[0, 0]
    a2 = sc_ref[0, 1]
    bo = sc_ref[0, 2]
    qq128 = jnp.concatenate([qil_ref[...], qcl_ref[...],
                             qih_ref[...], qch_ref[...]], axis=1)
    tq = jnp.dot(qq128, wbq_ref[...], preferred_element_type=jnp.float32,
                 precision=PREC) + b1_ref[...]
    kp = kp_ref[...]
    kp3 = kp.reshape(BB, T, 4 * D)
    qkp = (kp3 * qq128[:, None, :]).reshape(BB * T, 4 * D)
    h1 = (jnp.dot(kp, wbk_ref[...], preferred_element_type=jnp.float32,
                  precision=PREC)
          + jnp.dot(qkp, wbp_ref[...], preferred_element_type=jnp.float32,
                    precision=PREC))
    h1 = h1.reshape(BB, T, 160) + tq[:, None, :]
    h1 = jnp.where(h1 > 0, h1, a1 * h1).reshape(BB * T, 160)
    h2 = jnp.dot(h1, w2b_ref[...], preferred_element_type=jnp.float32,
                 precision=PREC) + b2_ref[...]
    h2 = jnp.where(h2 > 0, h2, a2 * h2)
    h23 = h2.reshape(BB, T, 80)
    s_lo = (jnp.sum(h23 * wol_ref[...][None, :, :], axis=2) + bo) * jnp.float32(0.125)
    s_hi = (jnp.sum(h23 * woh_ref[...][None, :, :], axis=2) + bo) * jnp.float32(0.125)
    lens_lo = lenl_ref[0, 0, :]
    lens_hi = lenh_ref[0, 0, :]
    jidx = lax.broadcasted_iota(jnp.int32, (BB, T), 1)
    neg = jnp.float32(-1e9)
    s_lo = jnp.where(jidx < lens_lo[:, None], s_lo, neg)
    s_hi = jnp.where(jidx < lens_hi[:, None], s_hi, neg)
    e_lo = jnp.exp(s_lo - jnp.max(s_lo, axis=1, keepdims=True))
    e_hi = jnp.exp(s_hi - jnp.max(s_hi, axis=1, keepdims=True))
    a_lo = e_lo / jnp.sum(e_lo, axis=1, keepdims=True)
    a_hi = e_hi / jnp.sum(e_hi, axis=1, keepdims=True)
    lane = lax.broadcasted_iota(jnp.int32, (BB, T, 4 * D), 2)
    att128 = jnp.where(lane < 2 * D, a_lo[:, :, None], a_hi[:, :, None])
    out_ref[...] = jnp.sum(kp3 * att128, axis=1)


def _attention(q_item, q_cate, kp, len3, wbq, b1_2, wbk, wbp, w2b, b2_2,
               wo_lo, wo_hi, sc):
    rep = lambda shape: pl.BlockSpec(shape, lambda i: tuple(0 for _ in shape))
    return pl.pallas_call(
        _att_body,
        grid=(NB2,),
        in_specs=[
            pl.BlockSpec((BB, D), lambda i: (i, 0)),
            pl.BlockSpec((BB, D), lambda i: (i, 0)),
            pl.BlockSpec((BB, D), lambda i: (i + NB2, 0)),
            pl.BlockSpec((BB, D), lambda i: (i + NB2, 0)),
            pl.BlockSpec((BB * T, 4 * D), lambda i: (i, 0)),
            pl.BlockSpec((1, 1, BB), lambda i: (i, 0, 0)),
            pl.BlockSpec((1, 1, BB), lambda i: (i + NB2, 0, 0)),
            rep((4 * D, 160)),
            rep((1, 160)),
            rep((4 * D, 160)),
            rep((4 * D, 160)),
            rep((160, 80)),
            rep((1, 80)),
            rep((1, 80)),
            rep((1, 80)),
            rep((1, 8)),
        ],
        out_specs=pl.BlockSpec((BB, 4 * D), lambda i: (i, 0)),
        out_shape=jax.ShapeDtypeStruct((B // 2, 4 * D), jnp.float32),
    )(q_item, q_cate, q_item, q_cate, kp, len3, len3,
      wbq, b1_2, wbk, wbp, w2b, b2_2, wo_lo, wo_hi, sc)


def _mlp_body(pr_ref, qi_ref, qc_ref, ao_ref, w1_ref, b1_ref, g1_ref, be1_ref,
              w2_ref, b2_ref, g2_ref, be2_ref, fw_ref, sc_ref, out_ref):
    a1 = sc_ref[0, 0]
    a2 = sc_ref[0, 1]
    fb = sc_ref[0, 2]
    ao = ao_ref[...]
    att = jnp.concatenate([ao[:, 0:2 * D], ao[:, 2 * D:4 * D]], axis=0)
    x = jnp.concatenate([pr_ref[...], qi_ref[...], qc_ref[...], att],
                        axis=1)
    h = jnp.dot(x, w1_ref[...], preferred_element_type=jnp.float32,
                precision=PREC) + b1_ref[...]
    mu = jnp.mean(h, axis=0, keepdims=True)
    var = jnp.mean((h - mu) ** 2, axis=0, keepdims=True)
    h = g1_ref[...] * (h - mu) / jnp.sqrt(var + 1e-5) + be1_ref[...]
    h = jnp.where(h > 0, h, a1 * h)
    h = jnp.dot(h, w2_ref[...], preferred_element_type=jnp.float32,
                precision=PREC) + b2_ref[...]
    mu = jnp.mean(h, axis=0, keepdims=True)
    var = jnp.mean((h - mu) ** 2, axis=0, keepdims=True)
    h = g2_ref[...] * (h - mu) / jnp.sqrt(var + 1e-5) + be2_ref[...]
    h = jnp.where(h > 0, h, a2 * h)
    o = jnp.dot(h, fw_ref[...], preferred_element_type=jnp.float32,
                precision=PREC) + fb
    out_ref[...] = jax.nn.sigmoid(o)


def _mlp(price2, q_item, q_cate, att_fold,
         w1, b1, g1, be1, w2, b2, g2, be2, fw, sc):
    return pl.pallas_call(
        _mlp_body,
        out_shape=jax.ShapeDtypeStruct((B, 1), jnp.float32),
    )(price2, q_item, q_cate, att_fold, w1, b1, g1, be1, w2, b2, g2, be2,
      fw, sc)


def kernel(price, item_id, cate_id, hist_item_id, hist_cate_id,
           hist_item_id_length, hist_cate_id_length,
           emb_item, emb_cate, emb_hist_item, emb_hist_cate,
           att_W1, att_b1, att_a1, att_W2, att_b2, att_a2, att_Wo, att_bo,
           mlp_W1, mlp_b1, mlp_g1, mlp_be1, mlp_a1,
           mlp_W2, mlp_b2, mlp_g2, mlp_be2, mlp_a2,
           fin_W, fin_b):
    q_item, q_cate = _sc_gather_q(emb_item, emb_cate, item_id, cate_id)
    kp = _sc_gather_hist(emb_hist_item, emb_hist_cate,
                         hist_item_id.reshape(-1), hist_cate_id.reshape(-1))

    wq = att_W1[0:2 * D] + att_W1[4 * D:6 * D]
    wk = att_W1[2 * D:4 * D] - att_W1[4 * D:6 * D]
    wp = att_W1[6 * D:8 * D]
    z64 = jnp.zeros((2 * D, 80), jnp.float32)
    z80 = jnp.zeros((80, 40), jnp.float32)
    wbq = jnp.block([[wq, z64], [z64, wq]])
    wbk = jnp.block([[wk, z64], [z64, wk]])
    wbp = jnp.block([[wp, z64], [z64, wp]])
    w2b = jnp.block([[att_W2, z80], [z80, att_W2]])
    b1_2 = jnp.concatenate([att_b1, att_b1]).reshape(1, 160)
    b2_2 = jnp.concatenate([att_b2, att_b2]).reshape(1, 80)
    wo = att_Wo.reshape(40)
    z40 = jnp.zeros((40,), jnp.float32)
    wo_lo = jnp.concatenate([wo, z40]).reshape(1, 80)
    wo_hi = jnp.concatenate([z40, wo]).reshape(1, 80)
    keys_len = jnp.minimum(hist_item_id_length, hist_cate_id_length)
    len3 = keys_len.astype(jnp.int32).reshape(B // BB, 1, BB)
    sc_att = jnp.stack([att_a1, att_a2, att_bo[0]] + [jnp.float32(0)] * 5)
    att_fold = _attention(q_item, q_cate, kp, len3, wbq, b1_2,
                          wbk, wbp, w2b, b2_2, wo_lo, wo_hi,
                          sc_att.reshape(1, 8))

    sc_mlp = jnp.stack([mlp_a1, mlp_a2, fin_b[0]] + [jnp.float32(0)] * 5)
    return _mlp(price.reshape(B, 1), q_item, q_cate, att_fold,
                mlp_W1, mlp_b1.reshape(1, 200), mlp_g1.reshape(1, 200),
                mlp_be1.reshape(1, 200), mlp_W2, mlp_b2.reshape(1, 80),
                mlp_g2.reshape(1, 80), mlp_be2.reshape(1, 80),
                fin_W, sc_mlp.reshape(1, 8))
